# baseline (speedup 1.0000x reference)
import functools

import jax
import jax.numpy as jnp
from jax import lax
from jax.experimental import pallas as pl
from jax.experimental.pallas import tpu as pltpu
from jax.experimental.pallas import tpu_sc as plsc

N = 10000
E = 320000
D = 128
G = 64
LAT = 64
N_PAD = 10240

CHUNK = 64
NC, NS = 2, 16
NW = NC * NS
ROWS_PER_TILE = N_PAD // NS
CPT = 160
NCHUNK = NW * CPT
E_PAD = NCHUNK * CHUNK
NBUF = 4
SEG = 40
NSEG = CPT // SEG

BE = 16000
BN = 1024


def _edge_mlp_body(eat_ref, w1t_ref, b1_ref, w2_ref, b2_ref, out_ref):
    h = jnp.dot(w1t_ref[...], eat_ref[...], preferred_element_type=jnp.float32)
    h = jnp.maximum(h + b1_ref[...], 0.0)
    out_ref[...] = (jnp.dot(w2_ref[...], h, preferred_element_type=jnp.float32)
                    + b2_ref[...])


def _edge_mlp(edge_attr_t, w1t, b1c, w2row, b2):
    out = pl.pallas_call(
        _edge_mlp_body,
        grid=(E // BE,),
        in_specs=[
            pl.BlockSpec((16, BE), lambda i: (0, i)),
            pl.BlockSpec((16, 16), lambda i: (0, 0)),
            pl.BlockSpec((16, 1), lambda i: (0, 0)),
            pl.BlockSpec((1, 16), lambda i: (0, 0)),
            pl.BlockSpec((1, 1), lambda i: (0, 0)),
        ],
        out_specs=pl.BlockSpec((1, BE), lambda i: (0, i)),
        out_shape=jax.ShapeDtypeStruct((1, E), jnp.float32),
    )(edge_attr_t, w1t, b1c, w2row, b2)
    return out.reshape(E)


def _scale1_body(x_ref, degp_ref, w_ref, y_ref, dinv_ref):
    deg = degp_ref[0, :] + degp_ref[1, :]
    dinv = jnp.where(deg > 0, lax.rsqrt(jnp.maximum(deg, 1e-12)), 0.0)
    xw = jnp.dot(x_ref[...], w_ref[...], preferred_element_type=jnp.float32)
    y_ref[...] = xw * dinv[:, None]
    dinv_ref[...] = dinv


def _scale1(x_p, degp, w):
    return pl.pallas_call(
        _scale1_body,
        grid=(N_PAD // BN,),
        in_specs=[
            pl.BlockSpec((BN, D), lambda i: (i, 0)),
            pl.BlockSpec((2, BN), lambda i: (0, i)),
            pl.BlockSpec((D, D), lambda i: (0, 0)),
        ],
        out_specs=[
            pl.BlockSpec((BN, D), lambda i: (i, 0)),
            pl.BlockSpec((BN,), lambda i: (i,)),
        ],
        out_shape=[
            jax.ShapeDtypeStruct((N_PAD, D), jnp.float32),
            jax.ShapeDtypeStruct((N_PAD,), jnp.float32),
        ],
    )(x_p, degp, w)


def _combine1_body(p_ref, y1_ref, dinv_ref, b_ref, w2_ref, y2_ref):
    dinv = dinv_ref[...]
    h = (p_ref[0] + p_ref[1] + y1_ref[...]) * dinv[:, None] + b_ref[...]
    h = jnp.maximum(h, 0.0)
    hw = jnp.dot(h, w2_ref[...], preferred_element_type=jnp.float32)
    y2_ref[...] = hw * dinv[:, None]


def _combine1(p1, y1, dinv, b, w2):
    return pl.pallas_call(
        _combine1_body,
        grid=(N_PAD // BN,),
        in_specs=[
            pl.BlockSpec((2, BN, D), lambda i: (0, i, 0)),
            pl.BlockSpec((BN, D), lambda i: (i, 0)),
            pl.BlockSpec((BN,), lambda i: (i,)),
            pl.BlockSpec((1, D), lambda i: (0, 0)),
            pl.BlockSpec((D, D), lambda i: (0, 0)),
        ],
        out_specs=pl.BlockSpec((BN, D), lambda i: (i, 0)),
        out_shape=jax.ShapeDtypeStruct((N_PAD, D), jnp.float32),
    )(p1, y1, dinv, b, w2)


def _final_body(p_ref, y2_ref, dinv_ref, b_ref, mask_ref, l1w_ref, l1b_ref,
                mw_ref, mb_ref, out_ref, acc_ref, cnt_ref):
    i = pl.program_id(0)

    @pl.when(i == 0)
    def _():
        acc_ref[...] = jnp.zeros_like(acc_ref)
        cnt_ref[...] = jnp.zeros_like(cnt_ref)

    dinv = dinv_ref[...]
    h = (p_ref[0] + p_ref[1] + y2_ref[...]) * dinv[:, None] + b_ref[...]
    h = jnp.maximum(h, 0.0)
    oh = (lax.broadcasted_iota(jnp.int32, (G, BN), 0)
          == mask_ref[...][None, :]).astype(jnp.float32)
    acc_ref[...] += jnp.dot(oh, h, preferred_element_type=jnp.float32)
    cnt_ref[...] += jnp.sum(oh, axis=1)

    @pl.when(i == pl.num_programs(0) - 1)
    def _():
        pooled = acc_ref[...] / jnp.maximum(cnt_ref[...], 1.0)[:, None]
        emb = jnp.dot(pooled, l1w_ref[...], preferred_element_type=jnp.float32)
        emb = jnp.maximum(emb + l1b_ref[...], 0.0)
        out_ref[...] = (jnp.dot(emb, mw_ref[...],
                                preferred_element_type=jnp.float32)
                        + mb_ref[...])


def _final(p2, y2, dinv, b, mask_p, l1w, l1b, mw, mb):
    return pl.pallas_call(
        _final_body,
        grid=(N_PAD // BN,),
        in_specs=[
            pl.BlockSpec((2, BN, D), lambda i: (0, i, 0)),
            pl.BlockSpec((BN, D), lambda i: (i, 0)),
            pl.BlockSpec((BN,), lambda i: (i,)),
            pl.BlockSpec((1, D), lambda i: (0, 0)),
            pl.BlockSpec((BN,), lambda i: (i,)),
            pl.BlockSpec((D, D), lambda i: (0, 0)),
            pl.BlockSpec((1, D), lambda i: (0, 0)),
            pl.BlockSpec((D, LAT), lambda i: (0, 0)),
            pl.BlockSpec((1, LAT), lambda i: (0, 0)),
        ],
        out_specs=pl.BlockSpec((G, LAT), lambda i: (0, 0)),
        out_shape=jax.ShapeDtypeStruct((G, LAT), jnp.float32),
        scratch_shapes=[
            pltpu.VMEM((G, D), jnp.float32),
            pltpu.VMEM((G,), jnp.float32),
        ],
    )(p2, y2, dinv, b, mask_p, l1w, l1b, mw, mb)


_SC_MESH = plsc.VectorSubcoreMesh(core_axis_name="c", subcore_axis_name="s")


def _deg_body(ew2d_hbm, col2d_hbm, onz_hbm, out_hbm, colbuf, ewbuf, deg_sp,
              sem):
    c = lax.axis_index("c")
    s = lax.axis_index("s")
    wid = s * NC + c
    b0 = s * ROWS_PER_TILE
    pltpu.sync_copy(onz_hbm.at[c, pl.ds(b0, ROWS_PER_TILE)],
                    deg_sp.at[pl.ds(b0, ROWS_PER_TILE)])
    base = wid * CPT
    pltpu.sync_copy(col2d_hbm.at[pl.ds(base, CPT)], colbuf)
    pltpu.sync_copy(ew2d_hbm.at[pl.ds(base, CPT)], ewbuf)
    plsc.subcore_barrier()

    def batch(bi, carry):
        def issue(i, cc):
            pltpu.async_copy(ewbuf.at[bi * 16 + i], deg_sp.at[colbuf.at[bi * 16 + i]],
                             sem, add=True)
            return cc

        lax.fori_loop(0, 16, issue, 0)

        def drain(i, cc):
            pltpu.make_async_copy(ewbuf.at[0], deg_sp.at[colbuf.at[0]], sem).wait()
            return cc

        lax.fori_loop(0, 16, drain, 0)
        return carry

    lax.fori_loop(0, CPT // 16, batch, 0)
    plsc.subcore_barrier()
    pltpu.sync_copy(deg_sp.at[pl.ds(b0, ROWS_PER_TILE)],
                    out_hbm.at[c, pl.ds(b0, ROWS_PER_TILE)])


_deg_kernel = pl.kernel(
    _deg_body,
    out_type=jax.ShapeDtypeStruct((NC, N_PAD), jnp.float32),
    mesh=_SC_MESH,
    scratch_types=[
        pltpu.VMEM((CPT, CHUNK), jnp.int32),
        pltpu.VMEM((CPT, CHUNK), jnp.float32),
        pltpu.VMEM_SHARED((N_PAD,), jnp.float32),
        pltpu.SemaphoreType.DMA,
    ],
)


def _msg_body(y_hbm, row2d_hbm, col2d_hbm, ew2d_hbm, z_hbm, out_hbm,
              rowbuf, colbuf, ewbuf, rows0, rows1, rows2, rows3,
              semg0, semg1, semg2, semg3, sems0, sems1, sems2, sems3,
              acc_sp):
    c = lax.axis_index("c")
    s = lax.axis_index("s")
    wid = s * NC + c
    b0 = s * ROWS_PER_TILE
    rows_bufs = [rows0, rows1, rows2, rows3]
    semgs = [semg0, semg1, semg2, semg3]
    semss = [sems0, sems1, sems2, sems3]

    pltpu.sync_copy(z_hbm.at[pl.ds(b0, ROWS_PER_TILE)],
                    acc_sp.at[pl.ds(b0, ROWS_PER_TILE)])
    base = wid * CPT
    plsc.subcore_barrier()

    def process(chunk_i, k):
        pltpu.make_async_copy(y_hbm.at[rowbuf.at[0]], rows_bufs[k],
                              semgs[k]).wait()
        rb = rows_bufs[k]

        def scale(g, cc):
            evec = ewbuf[chunk_i, pl.ds(g * 16, 16)]
            for e16 in range(16):
                sv = evec[e16]
                r = g * 16 + e16
                for t in range(8):
                    rb[r, pl.ds(t * 16, 16)] = rb[r, pl.ds(t * 16, 16)] * sv
            return cc

        lax.fori_loop(0, CHUNK // 16, scale, 0)
        pltpu.async_copy(rb, acc_sp.at[colbuf.at[chunk_i]], semss[k], add=True)

    def seg_loop(sg, carry):
        segc = base + sg * SEG
        pltpu.sync_copy(row2d_hbm.at[pl.ds(segc, SEG)], rowbuf)
        pltpu.sync_copy(col2d_hbm.at[pl.ds(segc, SEG)], colbuf)
        pltpu.sync_copy(ew2d_hbm.at[pl.ds(segc, SEG)], ewbuf)
        for k in range(NBUF):
            pltpu.async_copy(y_hbm.at[rowbuf.at[k]], rows_bufs[k], semgs[k])

        def body(j, cc):
            for k in range(NBUF):
                process(NBUF * j + k, k)
            for k in range(NBUF):
                pltpu.make_async_copy(rows_bufs[k], acc_sp.at[colbuf.at[0]],
                                      semss[k]).wait()
                pltpu.async_copy(y_hbm.at[rowbuf.at[NBUF * j + NBUF + k]],
                                 rows_bufs[k], semgs[k])
            return cc

        lax.fori_loop(0, SEG // NBUF - 1, body, 0)
        for k in range(NBUF):
            process(SEG - NBUF + k, k)
            pltpu.make_async_copy(rows_bufs[k], acc_sp.at[colbuf.at[0]],
                                  semss[k]).wait()
        return carry

    lax.fori_loop(0, NSEG, seg_loop, 0)
    plsc.subcore_barrier()
    pltpu.sync_copy(acc_sp.at[pl.ds(b0, ROWS_PER_TILE)],
                    out_hbm.at[c, pl.ds(b0, ROWS_PER_TILE)])


_msg_kernel = pl.kernel(
    _msg_body,
    out_type=jax.ShapeDtypeStruct((NC, N_PAD, D), jnp.float32),
    mesh=_SC_MESH,
    scratch_types=(
        [pltpu.VMEM((SEG, CHUNK), jnp.int32)] * 2
        + [pltpu.VMEM((SEG, CHUNK), jnp.float32)]
        + [pltpu.VMEM((CHUNK, D), jnp.float32)] * NBUF
        + [pltpu.SemaphoreType.DMA] * (2 * NBUF)
        + [pltpu.VMEM_SHARED((N_PAD, D), jnp.float32)]
    ),
)


def kernel(x, edge_index, edge_attr, batch_mask, nn_W1, nn_b1, nn_W2, nn_b2,
           conv1_W, conv1_b, conv2_W, conv2_b, lin1_W, lin1_b,
           lin_mu_W, lin_mu_b):
    row2d = jnp.pad(edge_index[0], (0, E_PAD - E)).reshape(NCHUNK, CHUNK)
    col2d = jnp.pad(edge_index[1], (0, E_PAD - E)).reshape(NCHUNK, CHUNK)
    x_p = jnp.pad(x, ((0, N_PAD - N), (0, 0)))
    mask_p = jnp.pad(batch_mask, (0, N_PAD - N), constant_values=G)
    onz = jnp.stack([jnp.ones((N_PAD,), jnp.float32),
                     jnp.zeros((N_PAD,), jnp.float32)])
    zeros2d = jnp.zeros((N_PAD, D), jnp.float32)

    ew = _edge_mlp(edge_attr.T, nn_W1.T, nn_b1.reshape(16, 1),
                   nn_W2.reshape(1, 16), nn_b2.reshape(1, 1))
    ew2d = jnp.pad(ew, (0, E_PAD - E)).reshape(NCHUNK, CHUNK)
    degp = _deg_kernel(ew2d, col2d, onz)
    y1, dinv = _scale1(x_p, degp, conv1_W)
    p1 = _msg_kernel(y1, row2d, col2d, ew2d, zeros2d)
    y2 = _combine1(p1, y1, dinv, conv1_b.reshape(1, D), conv2_W)
    p2 = _msg_kernel(y2, row2d, col2d, ew2d, zeros2d)
    mu = _final(p2, y2, dinv, conv2_b.reshape(1, D), mask_p,
                lin1_W, lin1_b.reshape(1, D), lin_mu_W, lin_mu_b.reshape(1, LAT))
    return mu


# speedup vs baseline: 1.0319x; 1.0319x over previous
import functools

import jax
import jax.numpy as jnp
from jax import lax
from jax.experimental import pallas as pl
from jax.experimental.pallas import tpu as pltpu
from jax.experimental.pallas import tpu_sc as plsc

N = 10000
E = 320000
D = 128
G = 64
LAT = 64
N_PAD = 10240

CHUNK = 64
NC, NS = 2, 16
NW = NC * NS
ROWS_PER_TILE = N_PAD // NS
CPT = 160
NCHUNK = NW * CPT
E_PAD = NCHUNK * CHUNK
NBUF = 4
SEG = 40
NSEG = CPT // SEG

BE = 16000
BN = 1024


def _edge_mlp_body(eat_ref, w1t_ref, b1_ref, w2_ref, b2_ref, out_ref):
    h = jnp.dot(w1t_ref[...], eat_ref[...], preferred_element_type=jnp.float32)
    h = jnp.maximum(h + b1_ref[...], 0.0)
    out_ref[...] = (jnp.dot(w2_ref[...], h, preferred_element_type=jnp.float32)
                    + b2_ref[...])


def _edge_mlp(edge_attr_t, w1t, b1c, w2row, b2):
    out = pl.pallas_call(
        _edge_mlp_body,
        grid=(E // BE,),
        in_specs=[
            pl.BlockSpec((16, BE), lambda i: (0, i)),
            pl.BlockSpec((16, 16), lambda i: (0, 0)),
            pl.BlockSpec((16, 1), lambda i: (0, 0)),
            pl.BlockSpec((1, 16), lambda i: (0, 0)),
            pl.BlockSpec((1, 1), lambda i: (0, 0)),
        ],
        out_specs=pl.BlockSpec((1, BE), lambda i: (0, i)),
        out_shape=jax.ShapeDtypeStruct((1, E), jnp.float32),
    )(edge_attr_t, w1t, b1c, w2row, b2)
    return out.reshape(E)


def _scale1_body(x_ref, degp_ref, w_ref, y_ref, dinv_ref):
    deg = degp_ref[0, :] + degp_ref[1, :]
    dinv = jnp.where(deg > 0, lax.rsqrt(jnp.maximum(deg, 1e-12)), 0.0)
    xw = jnp.dot(x_ref[...], w_ref[...], preferred_element_type=jnp.float32)
    y_ref[...] = xw * dinv[:, None]
    dinv_ref[...] = dinv


def _scale1(x_p, degp, w):
    return pl.pallas_call(
        _scale1_body,
        grid=(N_PAD // BN,),
        in_specs=[
            pl.BlockSpec((BN, D), lambda i: (i, 0)),
            pl.BlockSpec((2, BN), lambda i: (0, i)),
            pl.BlockSpec((D, D), lambda i: (0, 0)),
        ],
        out_specs=[
            pl.BlockSpec((BN, D), lambda i: (i, 0)),
            pl.BlockSpec((BN,), lambda i: (i,)),
        ],
        out_shape=[
            jax.ShapeDtypeStruct((N_PAD, D), jnp.float32),
            jax.ShapeDtypeStruct((N_PAD,), jnp.float32),
        ],
    )(x_p, degp, w)


def _combine1_body(p_ref, y1_ref, dinv_ref, b_ref, w2_ref, y2_ref):
    dinv = dinv_ref[...]
    h = (p_ref[0] + p_ref[1] + y1_ref[...]) * dinv[:, None] + b_ref[...]
    h = jnp.maximum(h, 0.0)
    hw = jnp.dot(h, w2_ref[...], preferred_element_type=jnp.float32)
    y2_ref[...] = hw * dinv[:, None]


def _combine1(p1, y1, dinv, b, w2):
    return pl.pallas_call(
        _combine1_body,
        grid=(N_PAD // BN,),
        in_specs=[
            pl.BlockSpec((2, BN, D), lambda i: (0, i, 0)),
            pl.BlockSpec((BN, D), lambda i: (i, 0)),
            pl.BlockSpec((BN,), lambda i: (i,)),
            pl.BlockSpec((1, D), lambda i: (0, 0)),
            pl.BlockSpec((D, D), lambda i: (0, 0)),
        ],
        out_specs=pl.BlockSpec((BN, D), lambda i: (i, 0)),
        out_shape=jax.ShapeDtypeStruct((N_PAD, D), jnp.float32),
    )(p1, y1, dinv, b, w2)


def _final_body(p_ref, y2_ref, dinv_ref, b_ref, mask_ref, l1w_ref, l1b_ref,
                mw_ref, mb_ref, out_ref, acc_ref, cnt_ref):
    i = pl.program_id(0)

    @pl.when(i == 0)
    def _():
        acc_ref[...] = jnp.zeros_like(acc_ref)
        cnt_ref[...] = jnp.zeros_like(cnt_ref)

    dinv = dinv_ref[...]
    h = (p_ref[0] + p_ref[1] + y2_ref[...]) * dinv[:, None] + b_ref[...]
    h = jnp.maximum(h, 0.0)
    oh = (lax.broadcasted_iota(jnp.int32, (G, BN), 0)
          == mask_ref[...][None, :]).astype(jnp.float32)
    acc_ref[...] += jnp.dot(oh, h, preferred_element_type=jnp.float32)
    cnt_ref[...] += jnp.sum(oh, axis=1)

    @pl.when(i == pl.num_programs(0) - 1)
    def _():
        pooled = acc_ref[...] / jnp.maximum(cnt_ref[...], 1.0)[:, None]
        emb = jnp.dot(pooled, l1w_ref[...], preferred_element_type=jnp.float32)
        emb = jnp.maximum(emb + l1b_ref[...], 0.0)
        out_ref[...] = (jnp.dot(emb, mw_ref[...],
                                preferred_element_type=jnp.float32)
                        + mb_ref[...])


def _final(p2, y2, dinv, b, mask_p, l1w, l1b, mw, mb):
    return pl.pallas_call(
        _final_body,
        grid=(N_PAD // BN,),
        in_specs=[
            pl.BlockSpec((2, BN, D), lambda i: (0, i, 0)),
            pl.BlockSpec((BN, D), lambda i: (i, 0)),
            pl.BlockSpec((BN,), lambda i: (i,)),
            pl.BlockSpec((1, D), lambda i: (0, 0)),
            pl.BlockSpec((BN,), lambda i: (i,)),
            pl.BlockSpec((D, D), lambda i: (0, 0)),
            pl.BlockSpec((1, D), lambda i: (0, 0)),
            pl.BlockSpec((D, LAT), lambda i: (0, 0)),
            pl.BlockSpec((1, LAT), lambda i: (0, 0)),
        ],
        out_specs=pl.BlockSpec((G, LAT), lambda i: (0, 0)),
        out_shape=jax.ShapeDtypeStruct((G, LAT), jnp.float32),
        scratch_shapes=[
            pltpu.VMEM((G, D), jnp.float32),
            pltpu.VMEM((G,), jnp.float32),
        ],
    )(p2, y2, dinv, b, mask_p, l1w, l1b, mw, mb)


_SC_MESH = plsc.VectorSubcoreMesh(core_axis_name="c", subcore_axis_name="s")


def _deg_body(ew2d_hbm, col2d_hbm, onz_hbm, out_hbm, colbuf, ewbuf, deg_sp,
              sem):
    c = lax.axis_index("c")
    s = lax.axis_index("s")
    wid = s * NC + c
    b0 = s * ROWS_PER_TILE
    pltpu.sync_copy(onz_hbm.at[c, pl.ds(b0, ROWS_PER_TILE)],
                    deg_sp.at[pl.ds(b0, ROWS_PER_TILE)])
    base = wid * CPT
    pltpu.sync_copy(col2d_hbm.at[pl.ds(base, CPT)], colbuf)
    pltpu.sync_copy(ew2d_hbm.at[pl.ds(base, CPT)], ewbuf)
    plsc.subcore_barrier()

    def batch(bi, carry):
        def issue(i, cc):
            pltpu.async_copy(ewbuf.at[bi * 16 + i], deg_sp.at[colbuf.at[bi * 16 + i]],
                             sem, add=True)
            return cc

        lax.fori_loop(0, 16, issue, 0)

        def drain(i, cc):
            pltpu.make_async_copy(ewbuf.at[0], deg_sp.at[colbuf.at[0]], sem).wait()
            return cc

        lax.fori_loop(0, 16, drain, 0)
        return carry

    lax.fori_loop(0, CPT // 16, batch, 0)
    plsc.subcore_barrier()
    pltpu.sync_copy(deg_sp.at[pl.ds(b0, ROWS_PER_TILE)],
                    out_hbm.at[c, pl.ds(b0, ROWS_PER_TILE)])


_deg_kernel = pl.kernel(
    _deg_body,
    out_type=jax.ShapeDtypeStruct((NC, N_PAD), jnp.float32),
    mesh=_SC_MESH,
    scratch_types=[
        pltpu.VMEM((CPT, CHUNK), jnp.int32),
        pltpu.VMEM((CPT, CHUNK), jnp.float32),
        pltpu.VMEM_SHARED((N_PAD,), jnp.float32),
        pltpu.SemaphoreType.DMA,
    ],
)


def _msg_body(y_hbm, row2d_hbm, col2d_hbm, ew2d_hbm, z_hbm, out_hbm,
              rowbuf, colbuf, ewbuf, rows0, rows1, rows2, rows3,
              semg0, semg1, semg2, semg3, sems0, sems1, sems2, sems3,
              acc_sp):
    c = lax.axis_index("c")
    s = lax.axis_index("s")
    wid = s * NC + c
    b0 = s * ROWS_PER_TILE
    rows_bufs = [rows0, rows1, rows2, rows3]
    semgs = [semg0, semg1, semg2, semg3]
    semss = [sems0, sems1, sems2, sems3]

    pltpu.sync_copy(z_hbm.at[pl.ds(b0, ROWS_PER_TILE)],
                    acc_sp.at[pl.ds(b0, ROWS_PER_TILE)])
    base = wid * CPT
    plsc.subcore_barrier()

    def process(chunk_i, k):
        pltpu.make_async_copy(y_hbm.at[rowbuf.at[0]], rows_bufs[k],
                              semgs[k]).wait()
        rb = rows_bufs[k]

        def scale(g, cc):
            evec = ewbuf[chunk_i, pl.ds(g * 16, 16)]
            for e16 in range(16):
                sv = evec[e16]
                r = g * 16 + e16
                for t in range(8):
                    rb[r, pl.ds(t * 16, 16)] = rb[r, pl.ds(t * 16, 16)] * sv
            return cc

        lax.fori_loop(0, CHUNK // 16, scale, 0)
        pltpu.sync_copy(rb, acc_sp.at[colbuf.at[chunk_i]], add=True)

    def seg_loop(sg, carry):
        segc = base + sg * SEG
        pltpu.sync_copy(row2d_hbm.at[pl.ds(segc, SEG)], rowbuf)
        pltpu.sync_copy(col2d_hbm.at[pl.ds(segc, SEG)], colbuf)
        pltpu.sync_copy(ew2d_hbm.at[pl.ds(segc, SEG)], ewbuf)
        for k in range(NBUF):
            pltpu.async_copy(y_hbm.at[rowbuf.at[k]], rows_bufs[k], semgs[k])

        def body(j, cc):
            for k in range(NBUF):
                process(NBUF * j + k, k)
                pltpu.async_copy(y_hbm.at[rowbuf.at[NBUF * j + NBUF + k]],
                                 rows_bufs[k], semgs[k])
            return cc

        lax.fori_loop(0, SEG // NBUF - 1, body, 0)
        for k in range(NBUF):
            process(SEG - NBUF + k, k)
        return carry

    lax.fori_loop(0, NSEG, seg_loop, 0)
    plsc.subcore_barrier()
    pltpu.sync_copy(acc_sp.at[pl.ds(b0, ROWS_PER_TILE)],
                    out_hbm.at[c, pl.ds(b0, ROWS_PER_TILE)])


_msg_kernel = pl.kernel(
    _msg_body,
    out_type=jax.ShapeDtypeStruct((NC, N_PAD, D), jnp.float32),
    mesh=_SC_MESH,
    scratch_types=(
        [pltpu.VMEM((SEG, CHUNK), jnp.int32)] * 2
        + [pltpu.VMEM((SEG, CHUNK), jnp.float32)]
        + [pltpu.VMEM((CHUNK, D), jnp.float32)] * NBUF
        + [pltpu.SemaphoreType.DMA] * (2 * NBUF)
        + [pltpu.VMEM_SHARED((N_PAD, D), jnp.float32)]
    ),
)


def kernel(x, edge_index, edge_attr, batch_mask, nn_W1, nn_b1, nn_W2, nn_b2,
           conv1_W, conv1_b, conv2_W, conv2_b, lin1_W, lin1_b,
           lin_mu_W, lin_mu_b):
    row2d = jnp.pad(edge_index[0], (0, E_PAD - E)).reshape(NCHUNK, CHUNK)
    col2d = jnp.pad(edge_index[1], (0, E_PAD - E)).reshape(NCHUNK, CHUNK)
    x_p = jnp.pad(x, ((0, N_PAD - N), (0, 0)))
    mask_p = jnp.pad(batch_mask, (0, N_PAD - N), constant_values=G)
    onz = jnp.stack([jnp.ones((N_PAD,), jnp.float32),
                     jnp.zeros((N_PAD,), jnp.float32)])
    zeros2d = jnp.zeros((N_PAD, D), jnp.float32)

    ew = _edge_mlp(edge_attr.T, nn_W1.T, nn_b1.reshape(16, 1),
                   nn_W2.reshape(1, 16), nn_b2.reshape(1, 1))
    ew2d = jnp.pad(ew, (0, E_PAD - E)).reshape(NCHUNK, CHUNK)
    degp = _deg_kernel(ew2d, col2d, onz)
    y1, dinv = _scale1(x_p, degp, conv1_W)
    p1 = _msg_kernel(y1, row2d, col2d, ew2d, zeros2d)
    y2 = _combine1(p1, y1, dinv, conv1_b.reshape(1, D), conv2_W)
    p2 = _msg_kernel(y2, row2d, col2d, ew2d, zeros2d)
    mu = _final(p2, y2, dinv, conv2_b.reshape(1, D), mask_p,
                lin1_W, lin1_b.reshape(1, D), lin_mu_W, lin_mu_b.reshape(1, LAT))
    return mu


# speedup vs baseline: 1.1251x; 1.0904x over previous
import functools

import jax
import jax.numpy as jnp
from jax import lax
from jax.experimental import pallas as pl
from jax.experimental.pallas import tpu as pltpu
from jax.experimental.pallas import tpu_sc as plsc

N = 10000
E = 320000
D = 128
G = 64
LAT = 64
N_PAD = 10240

CHUNK = 128
NC, NS = 2, 16
NW = NC * NS
ROWS_PER_TILE = N_PAD // NS
CPT = 80
NCHUNK = NW * CPT
E_PAD = NCHUNK * CHUNK
NBUF = 2
SEG = 16
NSEG = CPT // SEG

BE = 16000
BN = 1024


def _edge_mlp_body(eat_ref, w1t_ref, b1_ref, w2_ref, b2_ref, out_ref):
    h = jnp.dot(w1t_ref[...], eat_ref[...], preferred_element_type=jnp.float32)
    h = jnp.maximum(h + b1_ref[...], 0.0)
    out_ref[...] = (jnp.dot(w2_ref[...], h, preferred_element_type=jnp.float32)
                    + b2_ref[...])


def _edge_mlp(edge_attr_t, w1t, b1c, w2row, b2):
    out = pl.pallas_call(
        _edge_mlp_body,
        grid=(E // BE,),
        in_specs=[
            pl.BlockSpec((16, BE), lambda i: (0, i)),
            pl.BlockSpec((16, 16), lambda i: (0, 0)),
            pl.BlockSpec((16, 1), lambda i: (0, 0)),
            pl.BlockSpec((1, 16), lambda i: (0, 0)),
            pl.BlockSpec((1, 1), lambda i: (0, 0)),
        ],
        out_specs=pl.BlockSpec((1, BE), lambda i: (0, i)),
        out_shape=jax.ShapeDtypeStruct((1, E), jnp.float32),
    )(edge_attr_t, w1t, b1c, w2row, b2)
    return out.reshape(E)


def _scale1_body(x_ref, degp_ref, w_ref, y_ref, dinv_ref):
    deg = degp_ref[0, :] + degp_ref[1, :]
    dinv = jnp.where(deg > 0, lax.rsqrt(jnp.maximum(deg, 1e-12)), 0.0)
    xw = jnp.dot(x_ref[...], w_ref[...], preferred_element_type=jnp.float32)
    y_ref[...] = xw * dinv[:, None]
    dinv_ref[...] = dinv


def _scale1(x_p, degp, w):
    return pl.pallas_call(
        _scale1_body,
        grid=(N_PAD // BN,),
        in_specs=[
            pl.BlockSpec((BN, D), lambda i: (i, 0)),
            pl.BlockSpec((2, BN), lambda i: (0, i)),
            pl.BlockSpec((D, D), lambda i: (0, 0)),
        ],
        out_specs=[
            pl.BlockSpec((BN, D), lambda i: (i, 0)),
            pl.BlockSpec((BN,), lambda i: (i,)),
        ],
        out_shape=[
            jax.ShapeDtypeStruct((N_PAD, D), jnp.float32),
            jax.ShapeDtypeStruct((N_PAD,), jnp.float32),
        ],
    )(x_p, degp, w)


def _combine1_body(p_ref, y1_ref, dinv_ref, b_ref, w2_ref, y2_ref):
    dinv = dinv_ref[...]
    h = (p_ref[0] + p_ref[1] + y1_ref[...]) * dinv[:, None] + b_ref[...]
    h = jnp.maximum(h, 0.0)
    hw = jnp.dot(h, w2_ref[...], preferred_element_type=jnp.float32)
    y2_ref[...] = hw * dinv[:, None]


def _combine1(p1, y1, dinv, b, w2):
    return pl.pallas_call(
        _combine1_body,
        grid=(N_PAD // BN,),
        in_specs=[
            pl.BlockSpec((2, BN, D), lambda i: (0, i, 0)),
            pl.BlockSpec((BN, D), lambda i: (i, 0)),
            pl.BlockSpec((BN,), lambda i: (i,)),
            pl.BlockSpec((1, D), lambda i: (0, 0)),
            pl.BlockSpec((D, D), lambda i: (0, 0)),
        ],
        out_specs=pl.BlockSpec((BN, D), lambda i: (i, 0)),
        out_shape=jax.ShapeDtypeStruct((N_PAD, D), jnp.float32),
    )(p1, y1, dinv, b, w2)


def _final_body(p_ref, y2_ref, dinv_ref, b_ref, mask_ref, l1w_ref, l1b_ref,
                mw_ref, mb_ref, out_ref, acc_ref, cnt_ref):
    i = pl.program_id(0)

    @pl.when(i == 0)
    def _():
        acc_ref[...] = jnp.zeros_like(acc_ref)
        cnt_ref[...] = jnp.zeros_like(cnt_ref)

    dinv = dinv_ref[...]
    h = (p_ref[0] + p_ref[1] + y2_ref[...]) * dinv[:, None] + b_ref[...]
    h = jnp.maximum(h, 0.0)
    oh = (lax.broadcasted_iota(jnp.int32, (G, BN), 0)
          == mask_ref[...][None, :]).astype(jnp.float32)
    acc_ref[...] += jnp.dot(oh, h, preferred_element_type=jnp.float32)
    cnt_ref[...] += jnp.sum(oh, axis=1)

    @pl.when(i == pl.num_programs(0) - 1)
    def _():
        pooled = acc_ref[...] / jnp.maximum(cnt_ref[...], 1.0)[:, None]
        emb = jnp.dot(pooled, l1w_ref[...], preferred_element_type=jnp.float32)
        emb = jnp.maximum(emb + l1b_ref[...], 0.0)
        out_ref[...] = (jnp.dot(emb, mw_ref[...],
                                preferred_element_type=jnp.float32)
                        + mb_ref[...])


def _final(p2, y2, dinv, b, mask_p, l1w, l1b, mw, mb):
    return pl.pallas_call(
        _final_body,
        grid=(N_PAD // BN,),
        in_specs=[
            pl.BlockSpec((2, BN, D), lambda i: (0, i, 0)),
            pl.BlockSpec((BN, D), lambda i: (i, 0)),
            pl.BlockSpec((BN,), lambda i: (i,)),
            pl.BlockSpec((1, D), lambda i: (0, 0)),
            pl.BlockSpec((BN,), lambda i: (i,)),
            pl.BlockSpec((D, D), lambda i: (0, 0)),
            pl.BlockSpec((1, D), lambda i: (0, 0)),
            pl.BlockSpec((D, LAT), lambda i: (0, 0)),
            pl.BlockSpec((1, LAT), lambda i: (0, 0)),
        ],
        out_specs=pl.BlockSpec((G, LAT), lambda i: (0, 0)),
        out_shape=jax.ShapeDtypeStruct((G, LAT), jnp.float32),
        scratch_shapes=[
            pltpu.VMEM((G, D), jnp.float32),
            pltpu.VMEM((G,), jnp.float32),
        ],
    )(p2, y2, dinv, b, mask_p, l1w, l1b, mw, mb)


_SC_MESH = plsc.VectorSubcoreMesh(core_axis_name="c", subcore_axis_name="s")


def _deg_body(ew2d_hbm, col2d_hbm, onz_hbm, out_hbm, colbuf, ewbuf, deg_sp,
              sem):
    c = lax.axis_index("c")
    s = lax.axis_index("s")
    wid = s * NC + c
    b0 = s * ROWS_PER_TILE
    pltpu.sync_copy(onz_hbm.at[c, pl.ds(b0, ROWS_PER_TILE)],
                    deg_sp.at[pl.ds(b0, ROWS_PER_TILE)])
    base = wid * CPT
    pltpu.sync_copy(col2d_hbm.at[pl.ds(base, CPT)], colbuf)
    pltpu.sync_copy(ew2d_hbm.at[pl.ds(base, CPT)], ewbuf)
    plsc.subcore_barrier()

    def batch(bi, carry):
        def issue(i, cc):
            pltpu.async_copy(ewbuf.at[bi * 16 + i], deg_sp.at[colbuf.at[bi * 16 + i]],
                             sem, add=True)
            return cc

        lax.fori_loop(0, 16, issue, 0)

        def drain(i, cc):
            pltpu.make_async_copy(ewbuf.at[0], deg_sp.at[colbuf.at[0]], sem).wait()
            return cc

        lax.fori_loop(0, 16, drain, 0)
        return carry

    lax.fori_loop(0, CPT // 16, batch, 0)
    plsc.subcore_barrier()
    pltpu.sync_copy(deg_sp.at[pl.ds(b0, ROWS_PER_TILE)],
                    out_hbm.at[c, pl.ds(b0, ROWS_PER_TILE)])


_deg_kernel = pl.kernel(
    _deg_body,
    out_type=jax.ShapeDtypeStruct((NC, N_PAD), jnp.float32),
    mesh=_SC_MESH,
    scratch_types=[
        pltpu.VMEM((CPT, CHUNK), jnp.int32),
        pltpu.VMEM((CPT, CHUNK), jnp.float32),
        pltpu.VMEM_SHARED((N_PAD,), jnp.float32),
        pltpu.SemaphoreType.DMA,
    ],
)


def _msg_body(y_hbm, row2d_hbm, col2d_hbm, ew2d_hbm, z_hbm, out_hbm,
              rowbuf, colbuf, ewbuf, rows0, rows1, semg0, semg1,
              acc_sp):
    c = lax.axis_index("c")
    s = lax.axis_index("s")
    wid = s * NC + c
    b0 = s * ROWS_PER_TILE
    rows_bufs = [rows0, rows1]
    semgs = [semg0, semg1]

    pltpu.sync_copy(z_hbm.at[pl.ds(b0, ROWS_PER_TILE)],
                    acc_sp.at[pl.ds(b0, ROWS_PER_TILE)])
    base = wid * CPT
    plsc.subcore_barrier()

    def process(chunk_i, k):
        pltpu.make_async_copy(y_hbm.at[rowbuf.at[0]], rows_bufs[k],
                              semgs[k]).wait()
        rb = rows_bufs[k]

        def scale(g, cc):
            evec = ewbuf[chunk_i, pl.ds(g * 16, 16)]
            for e16 in range(16):
                sv = evec[e16]
                r = g * 16 + e16
                for t in range(8):
                    rb[r, pl.ds(t * 16, 16)] = rb[r, pl.ds(t * 16, 16)] * sv
            return cc

        lax.fori_loop(0, CHUNK // 16, scale, 0)
        pltpu.sync_copy(rb, acc_sp.at[colbuf.at[chunk_i]], add=True)

    def seg_loop(sg, carry):
        segc = base + sg * SEG
        pltpu.sync_copy(row2d_hbm.at[pl.ds(segc, SEG)], rowbuf)
        pltpu.sync_copy(col2d_hbm.at[pl.ds(segc, SEG)], colbuf)
        pltpu.sync_copy(ew2d_hbm.at[pl.ds(segc, SEG)], ewbuf)
        for k in range(NBUF):
            pltpu.async_copy(y_hbm.at[rowbuf.at[k]], rows_bufs[k], semgs[k])

        def body(j, cc):
            for k in range(NBUF):
                process(NBUF * j + k, k)
                pltpu.async_copy(y_hbm.at[rowbuf.at[NBUF * j + NBUF + k]],
                                 rows_bufs[k], semgs[k])
            return cc

        lax.fori_loop(0, SEG // NBUF - 1, body, 0)
        for k in range(NBUF):
            process(SEG - NBUF + k, k)
        return carry

    lax.fori_loop(0, NSEG, seg_loop, 0)
    plsc.subcore_barrier()
    pltpu.sync_copy(acc_sp.at[pl.ds(b0, ROWS_PER_TILE)],
                    out_hbm.at[c, pl.ds(b0, ROWS_PER_TILE)])


_msg_kernel = pl.kernel(
    _msg_body,
    out_type=jax.ShapeDtypeStruct((NC, N_PAD, D), jnp.float32),
    mesh=_SC_MESH,
    scratch_types=(
        [pltpu.VMEM((SEG, CHUNK), jnp.int32)] * 2
        + [pltpu.VMEM((SEG, CHUNK), jnp.float32)]
        + [pltpu.VMEM((CHUNK, D), jnp.float32)] * NBUF
        + [pltpu.SemaphoreType.DMA] * NBUF
        + [pltpu.VMEM_SHARED((N_PAD, D), jnp.float32)]
    ),
)


def kernel(x, edge_index, edge_attr, batch_mask, nn_W1, nn_b1, nn_W2, nn_b2,
           conv1_W, conv1_b, conv2_W, conv2_b, lin1_W, lin1_b,
           lin_mu_W, lin_mu_b):
    row2d = jnp.pad(edge_index[0], (0, E_PAD - E)).reshape(NCHUNK, CHUNK)
    col2d = jnp.pad(edge_index[1], (0, E_PAD - E)).reshape(NCHUNK, CHUNK)
    x_p = jnp.pad(x, ((0, N_PAD - N), (0, 0)))
    mask_p = jnp.pad(batch_mask, (0, N_PAD - N), constant_values=G)
    onz = jnp.stack([jnp.ones((N_PAD,), jnp.float32),
                     jnp.zeros((N_PAD,), jnp.float32)])
    zeros2d = jnp.zeros((N_PAD, D), jnp.float32)

    ew = _edge_mlp(edge_attr.T, nn_W1.T, nn_b1.reshape(16, 1),
                   nn_W2.reshape(1, 16), nn_b2.reshape(1, 1))
    ew2d = jnp.pad(ew, (0, E_PAD - E)).reshape(NCHUNK, CHUNK)
    degp = _deg_kernel(ew2d, col2d, onz)
    y1, dinv = _scale1(x_p, degp, conv1_W)
    p1 = _msg_kernel(y1, row2d, col2d, ew2d, zeros2d)
    y2 = _combine1(p1, y1, dinv, conv1_b.reshape(1, D), conv2_W)
    p2 = _msg_kernel(y2, row2d, col2d, ew2d, zeros2d)
    mu = _final(p2, y2, dinv, conv2_b.reshape(1, D), mask_p,
                lin1_W, lin1_b.reshape(1, D), lin_mu_W, lin_mu_b.reshape(1, LAT))
    return mu
